# Initial kernel scaffold; baseline (speedup 1.0000x reference)
import jax
import jax.numpy as jnp
from jax.experimental import pallas as pl


def kernel(data, W0, b0, g0, be0, W1, b1, g1, be1, Wp, bp):
    raise NotImplementedError("write your pallas kernel here")


# speedup vs baseline: 9.8620x; 9.8620x over previous
import functools

import jax
import jax.numpy as jnp
import numpy as np
from jax.experimental import pallas as pl
from jax.experimental.pallas import tpu as pltpu

_B, _N, _HID = 8, 400, 64
_DIN = _N * (_N - 1) // 2
_PADW = 80000
_LW = 512
_EPS = 1e-5
_P = jax.lax.Precision.HIGHEST


def _dot(a, b):
    return jax.lax.dot_general(a, b, (((1,), (0,)), ((), ())),
                               precision=_P, preferred_element_type=jnp.float32)


def _dotT(a, b):
    return jax.lax.dot_general(a, b, (((0,), (0,)), ((), ())),
                               precision=_P, preferred_element_type=jnp.float32)


def _bn(h, gamma, beta):
    d = h - h[0:1, :]
    mean_d = jnp.mean(d, axis=0, keepdims=True)
    dc = d - mean_d
    var = jnp.mean(dc * dc, axis=0, keepdims=True)
    return dc * jax.lax.rsqrt(var + _EPS) * gamma + beta


def _main_body(dpad_ref, W0_ref, b0_ref, g0_ref, be0_ref,
               W1_ref, b1_ref, g1_ref, be1_ref,
               out_ref, U_scr, h_scr, g_scr, dinv_scr):
    jcol = jax.lax.broadcasted_iota(jnp.int32, (_B, _LW), 1)

    def build(i, carry):
        u = i * (_N - 2) - (i * (i - 1)) // 2
        ua = (u // 128) * 128
        r = u - ua
        w640 = dpad_ref[:, pl.ds(ua, 640)]
        w = pltpu.roll(w640, -r, 1)[:, :_LW]
        row = jnp.where((jcol == i) | ((jcol > i) & (jcol < _N) & (w != 0.0)),
                        1.0, 0.0)
        U_scr[:, pl.ds(i, 1), :] = row[:, None, :]
        return carry

    jax.lax.fori_loop(0, _N, build, 0)

    ones_c = jnp.ones((_N, 1), jnp.float32)
    r_i = jax.lax.broadcasted_iota(jnp.int32, (_N, _N), 0)
    c_j = jax.lax.broadcasted_iota(jnp.int32, (_N, _N), 1)
    eye = jnp.where(r_i == c_j, 1.0, 0.0)
    W0row = W0_ref[...]

    def phase_a(g, carry):
        Ug = U_scr[g, :, :_N]
        Ut = jax.lax.dot_general(Ug, eye, (((0,), (0,)), ((), ())),
                                 preferred_element_type=jnp.float32)
        Ah = Ug + Ut - eye
        U_scr[g, :, :_N] = Ah
        deg = _dot(Ah, ones_c)
        dinv = jax.lax.rsqrt(deg)
        dinv_scr[g, :, :] = dinv
        s = dinv * _dot(Ah, dinv)
        h0 = jnp.maximum(s * W0row + b0_ref[...], 0.0)
        h_scr[pl.ds(g * _N, _N), :] = h0
        return carry

    jax.lax.fori_loop(0, _B, phase_a, 0)

    x0 = _bn(h_scr[...], g0_ref[...], be0_ref[...])
    g_scr[...] = _dot(x0, W1_ref[...])

    def phase_b(g, carry):
        Ah = U_scr[g, :, :_N]
        dinv = dinv_scr[g, :, :]
        Gd = dinv * g_scr[pl.ds(g * _N, _N), :]
        y = _dot(Ah, Gd)
        h1 = jnp.maximum(dinv * y + b1_ref[...], 0.0)
        h_scr[pl.ds(g * _N, _N), :] = h1
        return carry

    jax.lax.fori_loop(0, _B, phase_b, 0)

    out_ref[...] = _bn(h_scr[...], g1_ref[...], be1_ref[...])


def _pool_body(feats_ref, Wp_ref, bp_ref, out_ref):
    out_ref[...] = jnp.maximum(_dot(feats_ref[...], Wp_ref[...]) + bp_ref[...],
                               0.0)


def _forward(data, W0, b0, g0, be0, W1, b1, g1, be1, Wp, bp, interpret=False):
    f32 = jnp.float32
    dpad = jnp.zeros((_B, _PADW), f32).at[:, 1:1 + _DIN].set(data)
    vecs = [v.reshape(1, _HID).astype(f32) for v in (b0, g0, be0, b1, g1, be1, bp)]
    b0r, g0r, be0r, b1r, g1r, be1r, bpr = vecs

    x1 = pl.pallas_call(
        _main_body,
        out_shape=jax.ShapeDtypeStruct((_B * _N, _HID), f32),
        scratch_shapes=[
            pltpu.VMEM((_B, _N, _LW), f32),
            pltpu.VMEM((_B * _N, _HID), f32),
            pltpu.VMEM((_B * _N, _HID), f32),
            pltpu.VMEM((_B, _N, 1), f32),
        ],
        interpret=interpret,
    )(dpad, W0.astype(f32), b0r, g0r, be0r, W1.astype(f32), b1r, g1r, be1r)

    feats = x1.reshape(_B, _N * _HID)
    out = pl.pallas_call(
        _pool_body,
        out_shape=jax.ShapeDtypeStruct((_B, _HID), f32),
        interpret=interpret,
    )(feats, Wp.astype(f32), bpr)
    return out


def kernel(data, W0, b0, g0, be0, W1, b1, g1, be1, Wp, bp):
    return _forward(data, W0, b0, g0, be0, W1, b1, g1, be1, Wp, bp)


# speedup vs baseline: 14.9033x; 1.5112x over previous
import functools

import jax
import jax.numpy as jnp
import numpy as np
from jax.experimental import pallas as pl
from jax.experimental.pallas import tpu as pltpu

_B, _N, _HID = 8, 400, 64
_DIN = _N * (_N - 1) // 2
_PADW = 80000
_LW = 512
_EPS = 1e-5
_P = jax.lax.Precision.HIGHEST


def _dot(a, b, precision=None):
    return jax.lax.dot_general(a, b, (((1,), (0,)), ((), ())),
                               precision=precision,
                               preferred_element_type=jnp.float32)


def _dotT(a, b):
    return jax.lax.dot_general(a, b, (((0,), (0,)), ((), ())),
                               precision=_P, preferred_element_type=jnp.float32)


def _bn(h, gamma, beta):
    d = h - h[0:1, :]
    mean_d = jnp.mean(d, axis=0, keepdims=True)
    dc = d - mean_d
    var = jnp.mean(dc * dc, axis=0, keepdims=True)
    return dc * jax.lax.rsqrt(var + _EPS) * gamma + beta


def _main_body(dpad_ref, W0_ref, b0_ref, g0_ref, be0_ref,
               W1_ref, b1_ref, g1_ref, be1_ref,
               out_ref, U_scr, h_scr, g_scr, dinv_scr):
    jcol = jax.lax.broadcasted_iota(jnp.int32, (_B, _LW), 1)

    def build(i, carry):
        u = i * (_N - 2) - (i * (i - 1)) // 2
        ua = (u // 128) * 128
        r = u - ua
        w640 = dpad_ref[:, pl.ds(ua, 640)]
        w = pltpu.roll(w640, -r, 1)[:, :_LW]
        row = jnp.where((jcol == i) | ((jcol > i) & (jcol < _N) & (w != 0.0)),
                        1.0, 0.0)
        U_scr[:, pl.ds(i, 1), :] = row[:, None, :]
        return carry

    jax.lax.fori_loop(0, _N, build, 0, unroll=4)

    ones_c = jnp.ones((_N, 1), jnp.float32)
    r_i = jax.lax.broadcasted_iota(jnp.int32, (_N, _N), 0)
    c_j = jax.lax.broadcasted_iota(jnp.int32, (_N, _N), 1)
    eye = jnp.where(r_i == c_j, 1.0, 0.0)
    W0row = W0_ref[...]

    def phase_a(g, carry):
        Ug = U_scr[g, :, :_N]
        Ut = jax.lax.dot_general(Ug, eye, (((0,), (0,)), ((), ())),
                                 preferred_element_type=jnp.float32)
        Ah = Ug + Ut - eye
        U_scr[g, :, :_N] = Ah
        deg = _dot(Ah, ones_c, _P)
        dinv = jax.lax.rsqrt(deg)
        dinv_scr[g, :, :] = dinv
        s = dinv * _dot(Ah, dinv, _P)
        h0 = jnp.maximum(s * W0row + b0_ref[...], 0.0)
        h_scr[pl.ds(g * _N, _N), :] = h0
        return carry

    jax.lax.fori_loop(0, _B, phase_a, 0)

    x0 = _bn(h_scr[...], g0_ref[...], be0_ref[...])
    g_scr[...] = _dot(x0, W1_ref[...])

    def phase_b(g, carry):
        Ah = U_scr[g, :, :_N]
        dinv = dinv_scr[g, :, :]
        Gd = dinv * g_scr[pl.ds(g * _N, _N), :]
        y = _dot(Ah, Gd)
        h1 = jnp.maximum(dinv * y + b1_ref[...], 0.0)
        h_scr[pl.ds(g * _N, _N), :] = h1
        return carry

    jax.lax.fori_loop(0, _B, phase_b, 0)

    out_ref[...] = _bn(h_scr[...], g1_ref[...], be1_ref[...])


_KCH = 4


def _pool_body(feats_ref, Wp_ref, bp_ref, out_ref):
    k = pl.program_id(0)
    part = _dot(feats_ref[...], Wp_ref[...])

    @pl.when(k == 0)
    def _():
        out_ref[...] = part

    @pl.when(k > 0)
    def _():
        out_ref[...] += part

    @pl.when(k == _KCH - 1)
    def _():
        out_ref[...] = jnp.maximum(out_ref[...] + bp_ref[...], 0.0)


def _forward(data, W0, b0, g0, be0, W1, b1, g1, be1, Wp, bp, interpret=False):
    f32 = jnp.float32
    dpad = jnp.zeros((_B, _PADW), f32).at[:, 1:1 + _DIN].set(data)
    vecs = [v.reshape(1, _HID).astype(f32) for v in (b0, g0, be0, b1, g1, be1, bp)]
    b0r, g0r, be0r, b1r, g1r, be1r, bpr = vecs

    x1 = pl.pallas_call(
        _main_body,
        out_shape=jax.ShapeDtypeStruct((_B * _N, _HID), f32),
        scratch_shapes=[
            pltpu.VMEM((_B, _N, _LW), f32),
            pltpu.VMEM((_B * _N, _HID), f32),
            pltpu.VMEM((_B * _N, _HID), f32),
            pltpu.VMEM((_B, _N, 1), f32),
        ],
        interpret=interpret,
    )(dpad, W0.astype(f32), b0r, g0r, be0r, W1.astype(f32), b1r, g1r, be1r)

    feats = x1.reshape(_B, _N * _HID)
    kc = _N * _HID // _KCH
    out = pl.pallas_call(
        _pool_body,
        grid=(_KCH,),
        in_specs=[
            pl.BlockSpec((_B, kc), lambda k: (0, k)),
            pl.BlockSpec((kc, _HID), lambda k: (k, 0)),
            pl.BlockSpec((1, _HID), lambda k: (0, 0)),
        ],
        out_specs=pl.BlockSpec((_B, _HID), lambda k: (0, 0)),
        out_shape=jax.ShapeDtypeStruct((_B, _HID), f32),
        interpret=interpret,
    )(feats, Wp.astype(f32), bpr)
    return out


def kernel(data, W0, b0, g0, be0, W1, b1, g1, be1, Wp, bp):
    return _forward(data, W0, b0, g0, be0, W1, b1, g1, be1, Wp, bp)
